# baseline (speedup 1.0000x reference)
import functools

import jax
import jax.numpy as jnp
from jax import lax
from jax.experimental import pallas as pl
from jax.experimental.pallas import tpu as pltpu
from jax.experimental.pallas import tpu_sc as plsc

N = 10000
NP = 10240
E = 160000
FC = 128
NCORES = 2
NTILES = 16
EB = 128
KB = 2
EPT = -(-E // NTILES)
NBATCH = -(-(-(-EPT // EB)) // (2 * KB)) * (2 * KB)
NG = NBATCH // KB
NBX = NBATCH + 2 * KB
EPAD = NTILES * NBATCH * EB
ZR = 64
NROW = NP // NTILES
DW = 128

BN = 512


def _sc_mesh():
    return plsc.VectorSubcoreMesh(core_axis_name="c", subcore_axis_name="s")


@functools.cache
def _build_degree():
    @functools.partial(
        pl.kernel,
        out_type=jax.ShapeDtypeStruct((2, NP, DW), jnp.float32),
        mesh=_sc_mesh(),
        scratch_types=[
            pltpu.VMEM((NBATCH, EB), jnp.int32),
            pltpu.VMEM((EB, DW), jnp.float32),
            pltpu.VMEM_SHARED((NP, DW), jnp.float32),
        ],
    )
    def degree(idx_hbm, ones_hbm, zeros_hbm, out_hbm, idx_v, ones_v, acc):
        cid = lax.axis_index("c")
        sid = lax.axis_index("s")
        pltpu.sync_copy(idx_hbm.at[cid].at[sid].at[pl.ds(0, NBATCH)], idx_v)
        pltpu.sync_copy(ones_hbm, ones_v)
        pltpu.sync_copy(zeros_hbm, acc.at[pl.ds(sid * NROW, NROW)])
        plsc.subcore_barrier()

        def body(j, carry):
            pltpu.sync_copy(ones_v, acc.at[idx_v.at[j]], add=True)
            return carry

        lax.fori_loop(0, NBATCH, body, 0)
        plsc.subcore_barrier()
        pltpu.sync_copy(acc.at[pl.ds(sid * NROW, NROW)],
                        out_hbm.at[cid].at[pl.ds(sid * NROW, NROW)])

    return degree


@functools.cache
def _build_propagate(nck):
    cpc = nck // NCORES

    @functools.partial(
        pl.kernel,
        out_type=jax.ShapeDtypeStruct((nck, NP, FC), jnp.float32),
        mesh=_sc_mesh(),
        scratch_types=(
            [pltpu.VMEM((2, KB, EB), jnp.int32),
             pltpu.VMEM((2, KB, EB), jnp.int32)]
            + [pltpu.VMEM((EB, FC), jnp.float32) for _ in range(KB)]
            + [pltpu.VMEM((ZR, FC), jnp.float32)]
            + [pltpu.SemaphoreType.DMA for _ in range(5 + 2 * KB)]
            + [pltpu.VMEM_SHARED((NP, FC), jnp.float32)]
        ),
    )
    def prop(x_hbm, idx_hbm, zeros_hbm, out_hbm, src_b, dst_b, *rest):
        gbufs = rest[:KB]
        zbuf = rest[KB]
        isems = rest[KB + 1:KB + 5]
        gsems = rest[KB + 5:KB + 5 + KB]
        ssems = rest[KB + 5 + KB:KB + 5 + 2 * KB]
        zsem = rest[KB + 5 + 2 * KB]
        acc = rest[KB + 6 + 2 * KB]
        cid = lax.axis_index("c")
        sid = lax.axis_index("s")
        pltpu.sync_copy(zeros_hbm, zbuf)

        def idx_slices(g, slot):
            return (
                (idx_hbm.at[0].at[sid].at[pl.ds(g * KB, KB)],
                 src_b.at[slot], isems[2 * slot]),
                (idx_hbm.at[1].at[sid].at[pl.ds(g * KB, KB)],
                 dst_b.at[slot], isems[2 * slot + 1]),
            )

        def idx_start(g, slot):
            for s, d, m in idx_slices(g, slot):
                pltpu.async_copy(s, d, m)

        def idx_wait(g, slot):
            for s, d, m in idx_slices(g, slot):
                pltpu.make_async_copy(s, d, m).wait()

        def group(c, slot):
            gh = [pltpu.async_copy(x_hbm.at[c].at[src_b.at[slot].at[b]],
                                   gbufs[b], gsems[b]) for b in range(KB)]
            sh = []
            for b in range(KB):
                gh[b].wait()
                sh.append(pltpu.async_copy(gbufs[b],
                                           acc.at[dst_b.at[slot].at[b]],
                                           ssems[b], add=True))
            for b in range(KB):
                sh[b].wait()

        for cc in range(cpc):
            c = cid * cpc + cc
            zh = [pltpu.async_copy(
                zbuf, acc.at[pl.ds(sid * NROW + z * ZR, ZR)], zsem)
                for z in range(NROW // ZR)]
            for h in zh:
                h.wait()
            plsc.subcore_barrier()

            idx_start(0, 0)
            idx_start(1, 1)

            def body(j, carry):
                g0 = j * 2
                idx_wait(g0, 0)
                group(c, 0)
                idx_start(g0 + 2, 0)
                idx_wait(g0 + 1, 1)
                group(c, 1)
                idx_start(g0 + 3, 1)
                return carry

            lax.fori_loop(0, NG // 2, body, 0)
            idx_wait(NG, 0)
            idx_wait(NG + 1, 1)
            plsc.subcore_barrier()
            pltpu.sync_copy(acc.at[pl.ds(sid * NROW, NROW)],
                            out_hbm.at[c].at[pl.ds(sid * NROW, NROW)])
            plsc.subcore_barrier()

    return prop


def _degree(idx_all, ones1, zeros1):
    return _build_degree()(idx_all, ones1, zeros1)


def _propagate(x, idx_all, zeros2):
    return _build_propagate(x.shape[0])(x, idx_all, zeros2)


def _norms(deg_ref):
    ns = lax.rsqrt(jnp.clip(deg_ref[0, :], 1.0, None))
    nd = lax.rsqrt(jnp.clip(deg_ref[1, :], 1.0, None))
    return ns, nd


def _scale_body(deg_ref, x_ref, o_ref):
    ns, _ = _norms(deg_ref)
    xs = x_ref[...] * ns[:, None]
    o_ref[0] = xs[:, :FC]
    o_ref[1] = xs[:, FC:]


def _scale(deg, x):
    return pl.pallas_call(
        _scale_body,
        grid=(NP // BN,),
        in_specs=[
            pl.BlockSpec((2, BN), lambda i: (0, i)),
            pl.BlockSpec((BN, 2 * FC), lambda i: (i, 0)),
        ],
        out_specs=pl.BlockSpec((2, BN, FC), lambda i: (0, i, 0)),
        out_shape=jax.ShapeDtypeStruct((2, NP, FC), jnp.float32),
    )(deg, x)


def _leaky(x):
    return jnp.where(x >= 0, x, 0.2 * x)


def _m1_body(a_ref, w_ref, b_ref, deg_ref, o_ref):
    a = jnp.concatenate([a_ref[0], a_ref[1]], axis=1)
    y = jnp.dot(a, w_ref[...], preferred_element_type=jnp.float32)
    ns, nd = _norms(deg_ref)
    y = _leaky(y * nd[:, None] + b_ref[0, :][None, :]) * ns[:, None]
    o_ref[0] = y


def _m1(agg1, W1, b1, deg):
    return pl.pallas_call(
        _m1_body,
        grid=(NP // BN, 2048 // FC),
        in_specs=[
            pl.BlockSpec((2, BN, FC), lambda i, j: (0, i, 0)),
            pl.BlockSpec((2 * FC, FC), lambda i, j: (0, j)),
            pl.BlockSpec((1, FC), lambda i, j: (0, j)),
            pl.BlockSpec((2, BN), lambda i, j: (0, i)),
        ],
        out_specs=pl.BlockSpec((1, BN, FC), lambda i, j: (j, i, 0)),
        out_shape=jax.ShapeDtypeStruct((2048 // FC, NP, FC), jnp.float32),
    )(agg1, W1, b1, deg)


def _m2_body(a_ref, w_ref, b_ref, deg_ref, o_ref, acc_ref):
    k = pl.program_id(1)

    @pl.when(k == 0)
    def _():
        acc_ref[...] = jnp.zeros_like(acc_ref)

    acc_ref[...] += jnp.dot(a_ref[0], w_ref[0],
                            preferred_element_type=jnp.float32)

    @pl.when(k == 15)
    def _():
        _, nd = _norms(deg_ref)
        y = acc_ref[...] * nd[:, None] + b_ref[0, :][None, :]
        o_ref[...] = _leaky(y)


def _m2(agg2, W2r, b2, deg):
    return pl.pallas_call(
        _m2_body,
        grid=(NP // BN, 16),
        in_specs=[
            pl.BlockSpec((1, BN, FC), lambda i, k: (k, i, 0)),
            pl.BlockSpec((1, FC, 2048), lambda i, k: (k, 0, 0)),
            pl.BlockSpec((1, 2048), lambda i, k: (0, 0)),
            pl.BlockSpec((2, BN), lambda i, k: (0, i)),
        ],
        out_specs=pl.BlockSpec((BN, 2048), lambda i, k: (i, 0)),
        out_shape=jax.ShapeDtypeStruct((NP, 2048), jnp.float32),
        scratch_shapes=[pltpu.VMEM((BN, 2048), jnp.float32)],
    )(agg2, W2r, b2, deg)


def _m3_body(a_ref, w_ref, deg_ref, o_ref, acc_ref):
    k = pl.program_id(1)

    @pl.when(k == 0)
    def _():
        acc_ref[...] = jnp.zeros_like(acc_ref)

    acc_ref[...] += jnp.dot(a_ref[...], w_ref[...],
                            preferred_element_type=jnp.float32)

    @pl.when(k == 3)
    def _():
        ns, _ = _norms(deg_ref)
        y = acc_ref[...] * ns[:, None]
        o_ref[0] = y[:, :FC]
        o_ref[1] = y[:, FC:]


def _m3(h2, W3, deg):
    bk = 512
    return pl.pallas_call(
        _m3_body,
        grid=(NP // BN, 2048 // bk),
        in_specs=[
            pl.BlockSpec((BN, bk), lambda i, k: (i, k)),
            pl.BlockSpec((bk, 2 * FC), lambda i, k: (k, 0)),
            pl.BlockSpec((2, BN), lambda i, k: (0, i)),
        ],
        out_specs=pl.BlockSpec((2, BN, FC), lambda i, k: (0, i, 0)),
        out_shape=jax.ShapeDtypeStruct((2, NP, FC), jnp.float32),
        scratch_shapes=[pltpu.VMEM((BN, 2 * FC), jnp.float32)],
    )(h2, W3, deg)


def _fin_body(a_ref, deg_ref, b_ref, o_ref):
    h = jnp.concatenate([a_ref[0], a_ref[1]], axis=1)
    _, nd = _norms(deg_ref)
    y = h * nd[:, None] + b_ref[...]
    nrm = jnp.sqrt(jnp.sum(y * y, axis=1, keepdims=True))
    o_ref[...] = y / jnp.clip(nrm, 1e-12, None)


def _fin(agg3, deg, b3):
    return pl.pallas_call(
        _fin_body,
        grid=(NP // BN,),
        in_specs=[
            pl.BlockSpec((2, BN, FC), lambda i: (0, i, 0)),
            pl.BlockSpec((2, BN), lambda i: (0, i)),
            pl.BlockSpec((1, 2 * FC), lambda i: (0, 0)),
        ],
        out_specs=pl.BlockSpec((BN, 2 * FC), lambda i: (i, 0)),
        out_shape=jax.ShapeDtypeStruct((NP, 2 * FC), jnp.float32),
    )(agg3, deg, b3)


def kernel(g, inputs, W1, b1, W2, b2, W3, b3):
    idx = jnp.pad(jnp.stack([g[0], g[1]]), ((0, 0), (0, EPAD - E)),
                  constant_values=N).reshape(2, NTILES, NBATCH, EB)
    idx_all = jnp.concatenate(
        [idx, jnp.full((2, NTILES, 2 * KB, EB), N, jnp.int32)], axis=2)
    x_pad = jnp.concatenate(
        [inputs, jnp.zeros((NP - N, 2 * FC), jnp.float32)])
    zeros2 = jnp.zeros((ZR, FC), jnp.float32)
    zeros1 = jnp.zeros((NROW, DW), jnp.float32)
    ones1 = jnp.ones((EB, DW), jnp.float32)

    deg = _degree(idx_all, ones1, zeros1)[:, :, 0]
    x1s = _scale(deg, x_pad)
    agg1 = _propagate(x1s, idx_all, zeros2)
    h1s = _m1(agg1, W1, b1.reshape(1, -1), deg)
    agg2 = _propagate(h1s, idx_all, zeros2)
    h2 = _m2(agg2, W2.reshape(16, FC, 2048),
             b2.reshape(1, -1), deg)
    t = _m3(h2, W3, deg)
    agg3 = _propagate(t, idx_all, zeros2)
    return _fin(agg3, deg, b3.reshape(1, -1))[:N]


# speedup vs baseline: 1.3508x; 1.3508x over previous
import functools

import jax
import jax.numpy as jnp
from jax import lax
from jax.experimental import pallas as pl
from jax.experimental.pallas import tpu as pltpu
from jax.experimental.pallas import tpu_sc as plsc

N = 10000
NP = 10240
E = 160000
FC = 128
NCORES = 2
NTILES = 16
EB = 128
EPT = -(-E // NTILES)
NBATCH = -(-EPT // EB)
EPAD = NTILES * NBATCH * EB
ZR = 64
NROW = NP // NTILES
DW = 128

BN = 512


def _sc_mesh():
    return plsc.VectorSubcoreMesh(core_axis_name="c", subcore_axis_name="s")


@functools.cache
def _build_degree():
    @functools.partial(
        pl.kernel,
        out_type=jax.ShapeDtypeStruct((2, NP, DW), jnp.float32),
        mesh=_sc_mesh(),
        scratch_types=[
            pltpu.VMEM((NBATCH, EB), jnp.int32),
            pltpu.VMEM((EB, DW), jnp.float32),
            pltpu.VMEM_SHARED((NP, DW), jnp.float32),
        ],
    )
    def degree(idx_hbm, ones_hbm, zeros_hbm, out_hbm, idx_v, ones_v, acc):
        cid = lax.axis_index("c")
        sid = lax.axis_index("s")
        pltpu.sync_copy(idx_hbm.at[cid].at[sid], idx_v)
        pltpu.sync_copy(ones_hbm, ones_v)
        pltpu.sync_copy(zeros_hbm, acc.at[pl.ds(sid * NROW, NROW)])
        plsc.subcore_barrier()

        def body(j, carry):
            pltpu.sync_copy(ones_v, acc.at[idx_v.at[j]], add=True)
            return carry

        lax.fori_loop(0, NBATCH, body, 0)
        plsc.subcore_barrier()
        pltpu.sync_copy(acc.at[pl.ds(sid * NROW, NROW)],
                        out_hbm.at[cid].at[pl.ds(sid * NROW, NROW)])

    return degree


@functools.cache
def _build_propagate(nck):
    cpc = nck // NCORES

    @functools.partial(
        pl.kernel,
        out_type=jax.ShapeDtypeStruct((nck, NP, FC), jnp.float32),
        mesh=_sc_mesh(),
        scratch_types=[
            pltpu.VMEM((NBATCH, EB), jnp.int32),
            pltpu.VMEM((NBATCH, EB), jnp.int32),
            pltpu.VMEM((EB, FC), jnp.float32),
            pltpu.VMEM((ZR, FC), jnp.float32),
            pltpu.SemaphoreType.DMA,
            pltpu.VMEM_SHARED((NP, FC), jnp.float32),
        ],
    )
    def prop(x_hbm, idx_hbm, zeros_hbm, out_hbm, src_v, dst_v, gbuf, zbuf,
             zsem, acc):
        cid = lax.axis_index("c")
        sid = lax.axis_index("s")
        pltpu.sync_copy(idx_hbm.at[0].at[sid], src_v)
        pltpu.sync_copy(idx_hbm.at[1].at[sid], dst_v)
        pltpu.sync_copy(zeros_hbm, zbuf)
        for cc in range(cpc):
            c = cid * cpc + cc
            zh = [pltpu.async_copy(
                zbuf, acc.at[pl.ds(sid * NROW + z * ZR, ZR)], zsem)
                for z in range(NROW // ZR)]
            for h in zh:
                h.wait()
            plsc.subcore_barrier()

            def body(j, carry):
                pltpu.sync_copy(x_hbm.at[c].at[src_v.at[j]], gbuf)
                pltpu.sync_copy(gbuf, acc.at[dst_v.at[j]], add=True)
                return carry

            lax.fori_loop(0, NBATCH, body, 0)
            plsc.subcore_barrier()
            pltpu.sync_copy(acc.at[pl.ds(sid * NROW, NROW)],
                            out_hbm.at[c].at[pl.ds(sid * NROW, NROW)])
            plsc.subcore_barrier()

    return prop


def _degree(idx_all, ones1, zeros1):
    return _build_degree()(idx_all, ones1, zeros1)


def _propagate(x, idx_all, zeros2):
    return _build_propagate(x.shape[0])(x, idx_all, zeros2)


def _norms(deg_ref):
    ns = lax.rsqrt(jnp.clip(deg_ref[0, :], 1.0, None))
    nd = lax.rsqrt(jnp.clip(deg_ref[1, :], 1.0, None))
    return ns, nd


def _scale_body(deg_ref, x_ref, o_ref):
    ns, _ = _norms(deg_ref)
    xs = x_ref[...] * ns[:, None]
    o_ref[0] = xs[:, :FC]
    o_ref[1] = xs[:, FC:]


def _scale(deg, x):
    return pl.pallas_call(
        _scale_body,
        grid=(NP // BN,),
        in_specs=[
            pl.BlockSpec((2, BN), lambda i: (0, i)),
            pl.BlockSpec((BN, 2 * FC), lambda i: (i, 0)),
        ],
        out_specs=pl.BlockSpec((2, BN, FC), lambda i: (0, i, 0)),
        out_shape=jax.ShapeDtypeStruct((2, NP, FC), jnp.float32),
    )(deg, x)


def _leaky(x):
    return jnp.where(x >= 0, x, 0.2 * x)


def _m1_body(a_ref, w_ref, b_ref, deg_ref, o_ref):
    a = jnp.concatenate([a_ref[0], a_ref[1]], axis=1)
    y = jnp.dot(a, w_ref[...], preferred_element_type=jnp.float32)
    _, nd = _norms(deg_ref)
    o_ref[...] = _leaky(y * nd[:, None] + b_ref[0, :][None, :])


def _m1(agg1, W1, b1, deg):
    return pl.pallas_call(
        _m1_body,
        grid=(NP // BN, 2048 // FC),
        in_specs=[
            pl.BlockSpec((2, BN, FC), lambda i, j: (0, i, 0)),
            pl.BlockSpec((2 * FC, FC), lambda i, j: (0, j)),
            pl.BlockSpec((1, FC), lambda i, j: (0, j)),
            pl.BlockSpec((2, BN), lambda i, j: (0, i)),
        ],
        out_specs=pl.BlockSpec((BN, FC), lambda i, j: (i, j)),
        out_shape=jax.ShapeDtypeStruct((NP, 2048), jnp.float32),
    )(agg1, W1, b1, deg)


BQ = 512


def _mm2_body(h_ref, w_ref, deg_ref, o_ref, acc_ref):
    k = pl.program_id(1)

    @pl.when(k == 0)
    def _():
        acc_ref[...] = jnp.zeros_like(acc_ref)

    acc_ref[...] += jnp.dot(h_ref[...], w_ref[...],
                            preferred_element_type=jnp.float32)

    @pl.when(k == 3)
    def _():
        ns, _ = _norms(deg_ref)
        y = acc_ref[...] * ns[:, None]
        for c in range(BQ // FC):
            o_ref[c] = y[:, c * FC:(c + 1) * FC]


def _mm2(h1, W2q, deg):
    return pl.pallas_call(
        _mm2_body,
        grid=(NP // BN, 2048 // BQ),
        in_specs=[
            pl.BlockSpec((BN, BQ), lambda i, k: (i, k)),
            pl.BlockSpec((BQ, BQ), lambda i, k: (k, 0)),
            pl.BlockSpec((2, BN), lambda i, k: (0, i)),
        ],
        out_specs=pl.BlockSpec((BQ // FC, BN, FC), lambda i, k: (0, i, 0)),
        out_shape=jax.ShapeDtypeStruct((BQ // FC, NP, FC), jnp.float32),
        scratch_shapes=[pltpu.VMEM((BN, BQ), jnp.float32)],
    )(h1, W2q, deg)


def _tacc_body(t_ref, a_ref, w_ref, b_ref, deg_ref, o_ref):
    h = jnp.concatenate([a_ref[c] for c in range(BQ // FC)], axis=1)
    ns, nd = _norms(deg_ref)
    h2 = _leaky(h * nd[:, None] + b_ref[0, :][None, :])
    y = jnp.dot(h2, w_ref[...], preferred_element_type=jnp.float32)
    y = y * ns[:, None]
    o_ref[0] = t_ref[0] + y[:, :FC]
    o_ref[1] = t_ref[1] + y[:, FC:]


def _tacc(t, agg4, W3q, b2q, deg):
    return pl.pallas_call(
        _tacc_body,
        grid=(NP // BN,),
        in_specs=[
            pl.BlockSpec((2, BN, FC), lambda i: (0, i, 0)),
            pl.BlockSpec((BQ // FC, BN, FC), lambda i: (0, i, 0)),
            pl.BlockSpec((BQ, 2 * FC), lambda i: (0, 0)),
            pl.BlockSpec((1, BQ), lambda i: (0, 0)),
            pl.BlockSpec((2, BN), lambda i: (0, i)),
        ],
        out_specs=pl.BlockSpec((2, BN, FC), lambda i: (0, i, 0)),
        out_shape=jax.ShapeDtypeStruct((2, NP, FC), jnp.float32),
        input_output_aliases={0: 0},
    )(t, agg4, W3q, b2q, deg)


def _fin_body(a_ref, deg_ref, b_ref, o_ref):
    h = jnp.concatenate([a_ref[0], a_ref[1]], axis=1)
    _, nd = _norms(deg_ref)
    y = h * nd[:, None] + b_ref[...]
    nrm = jnp.sqrt(jnp.sum(y * y, axis=1, keepdims=True))
    o_ref[...] = y / jnp.clip(nrm, 1e-12, None)


def _fin(agg3, deg, b3):
    return pl.pallas_call(
        _fin_body,
        grid=(NP // BN,),
        in_specs=[
            pl.BlockSpec((2, BN, FC), lambda i: (0, i, 0)),
            pl.BlockSpec((2, BN), lambda i: (0, i)),
            pl.BlockSpec((1, 2 * FC), lambda i: (0, 0)),
        ],
        out_specs=pl.BlockSpec((BN, 2 * FC), lambda i: (i, 0)),
        out_shape=jax.ShapeDtypeStruct((NP, 2 * FC), jnp.float32),
    )(agg3, deg, b3)


def kernel(g, inputs, W1, b1, W2, b2, W3, b3):
    idx_all = jnp.pad(jnp.stack([g[0], g[1]]), ((0, 0), (0, EPAD - E)),
                      constant_values=N).reshape(2, NTILES, NBATCH, EB)
    x_pad = jnp.concatenate(
        [inputs, jnp.zeros((NP - N, 2 * FC), jnp.float32)])
    zeros2 = jnp.zeros((ZR, FC), jnp.float32)
    zeros1 = jnp.zeros((NROW, DW), jnp.float32)
    ones1 = jnp.ones((EB, DW), jnp.float32)

    deg = _degree(idx_all, ones1, zeros1)[:, :, 0]
    x1s = _scale(deg, x_pad)
    agg1 = _propagate(x1s, idx_all, zeros2)
    h1 = _m1(agg1, W1, b1.reshape(1, -1), deg)
    aggs = []
    for q in range(4):
        mmq = _mm2(h1, W2[:, q * BQ:(q + 1) * BQ], deg)
        aggs.append(_propagate(mmq, idx_all, zeros2))
    t = jnp.zeros((2, NP, FC), jnp.float32)
    for q in range(4):
        t = _tacc(t, aggs[q], W3[q * BQ:(q + 1) * BQ],
                  b2[q * BQ:(q + 1) * BQ].reshape(1, -1), deg)
    agg3 = _propagate(t, idx_all, zeros2)
    return _fin(agg3, deg, b3.reshape(1, -1))[:N]


# speedup vs baseline: 1.3515x; 1.0005x over previous
import functools

import jax
import jax.numpy as jnp
from jax import lax
from jax.experimental import pallas as pl
from jax.experimental.pallas import tpu as pltpu
from jax.experimental.pallas import tpu_sc as plsc

N = 10000
NP = 10240
E = 160000
FC = 128
NCORES = 2
NTILES = 16
EB = 128
EPT = -(-E // NTILES)
NBATCH = -(-EPT // EB)
EPAD = NTILES * NBATCH * EB
ZR = 64
NROW = NP // NTILES
DW = 128

BN = 512


def _sc_mesh():
    return plsc.VectorSubcoreMesh(core_axis_name="c", subcore_axis_name="s")


@functools.cache
def _build_degree():
    @functools.partial(
        pl.kernel,
        out_type=jax.ShapeDtypeStruct((2, NP, DW), jnp.float32),
        mesh=_sc_mesh(),
        scratch_types=[
            pltpu.VMEM((NBATCH, EB), jnp.int32),
            pltpu.VMEM((EB, DW), jnp.float32),
            pltpu.VMEM_SHARED((NP, DW), jnp.float32),
        ],
    )
    def degree(idx_hbm, ones_hbm, zeros_hbm, out_hbm, idx_v, ones_v, acc):
        cid = lax.axis_index("c")
        sid = lax.axis_index("s")
        pltpu.sync_copy(idx_hbm.at[cid].at[sid], idx_v)
        pltpu.sync_copy(ones_hbm, ones_v)
        pltpu.sync_copy(zeros_hbm, acc.at[pl.ds(sid * NROW, NROW)])
        plsc.subcore_barrier()

        def body(j, carry):
            pltpu.sync_copy(ones_v, acc.at[idx_v.at[j]], add=True)
            return carry

        lax.fori_loop(0, NBATCH, body, 0)
        plsc.subcore_barrier()
        pltpu.sync_copy(acc.at[pl.ds(sid * NROW, NROW)],
                        out_hbm.at[cid].at[pl.ds(sid * NROW, NROW)])

    return degree


@functools.cache
def _build_propagate(nck):
    cpc = nck // NCORES

    @functools.partial(
        pl.kernel,
        out_type=jax.ShapeDtypeStruct((nck, NP, FC), jnp.float32),
        mesh=_sc_mesh(),
        scratch_types=[
            pltpu.VMEM((NBATCH, EB), jnp.int32),
            pltpu.VMEM((NBATCH, EB), jnp.int32),
            pltpu.VMEM((EB, FC), jnp.float32),
            pltpu.VMEM((ZR, FC), jnp.float32),
            pltpu.SemaphoreType.DMA,
            pltpu.VMEM_SHARED((NP, FC), jnp.float32),
        ],
    )
    def prop(x_hbm, idx_hbm, zeros_hbm, out_hbm, src_v, dst_v, gbuf, zbuf,
             zsem, acc):
        cid = lax.axis_index("c")
        sid = lax.axis_index("s")
        pltpu.sync_copy(idx_hbm.at[0].at[sid], src_v)
        pltpu.sync_copy(idx_hbm.at[1].at[sid], dst_v)
        pltpu.sync_copy(zeros_hbm, zbuf)
        for cc in range(cpc):
            c = cid * cpc + cc
            zh = [pltpu.async_copy(
                zbuf, acc.at[pl.ds(sid * NROW + z * ZR, ZR)], zsem)
                for z in range(NROW // ZR)]
            for h in zh:
                h.wait()
            plsc.subcore_barrier()

            def body(j, carry):
                pltpu.sync_copy(x_hbm.at[c].at[src_v.at[j]], gbuf)
                pltpu.sync_copy(gbuf, acc.at[dst_v.at[j]], add=True)
                return carry

            lax.fori_loop(0, NBATCH, body, 0)
            plsc.subcore_barrier()
            pltpu.sync_copy(acc.at[pl.ds(sid * NROW, NROW)],
                            out_hbm.at[c].at[pl.ds(sid * NROW, NROW)])
            plsc.subcore_barrier()

    return prop


def _degree(idx_all, ones1, zeros1):
    return _build_degree()(idx_all, ones1, zeros1)


def _propagate(x, idx_all, zeros2):
    return _build_propagate(x.shape[0])(x, idx_all, zeros2)


def _edge_arrays(g):
    idx_all = jnp.pad(jnp.stack([g[0], g[1]]), ((0, 0), (0, EPAD - E)),
                      constant_values=N).reshape(2, NTILES, NBATCH, EB)
    return idx_all


def _norms(deg_ref):
    ns = lax.rsqrt(jnp.clip(deg_ref[0, :], 1.0, None))
    nd = lax.rsqrt(jnp.clip(deg_ref[1, :], 1.0, None))
    return ns, nd


def _scale_body(deg_ref, x_ref, o_ref):
    ns, _ = _norms(deg_ref)
    xs = x_ref[...] * ns[:, None]
    o_ref[0] = xs[:, :FC]
    o_ref[1] = xs[:, FC:]


def _scale(deg, x):
    return pl.pallas_call(
        _scale_body,
        grid=(NP // BN,),
        in_specs=[
            pl.BlockSpec((2, BN), lambda i: (0, i)),
            pl.BlockSpec((BN, 2 * FC), lambda i: (i, 0)),
        ],
        out_specs=pl.BlockSpec((2, BN, FC), lambda i: (0, i, 0)),
        out_shape=jax.ShapeDtypeStruct((2, NP, FC), jnp.float32),
    )(deg, x)


def _leaky(x):
    return jnp.where(x >= 0, x, 0.2 * x)


def _m1_body(a_ref, w_ref, b_ref, deg_ref, o_ref):
    a = jnp.concatenate([a_ref[0], a_ref[1]], axis=1)
    y = jnp.dot(a, w_ref[...], preferred_element_type=jnp.float32)
    _, nd = _norms(deg_ref)
    o_ref[...] = _leaky(y * nd[:, None] + b_ref[0, :][None, :])


def _m1(agg1, W1, b1, deg):
    return pl.pallas_call(
        _m1_body,
        grid=(NP // BN, 2048 // FC),
        in_specs=[
            pl.BlockSpec((2, BN, FC), lambda i, j: (0, i, 0)),
            pl.BlockSpec((2 * FC, FC), lambda i, j: (0, j)),
            pl.BlockSpec((1, FC), lambda i, j: (0, j)),
            pl.BlockSpec((2, BN), lambda i, j: (0, i)),
        ],
        out_specs=pl.BlockSpec((BN, FC), lambda i, j: (i, j)),
        out_shape=jax.ShapeDtypeStruct((NP, 2048), jnp.float32),
    )(agg1, W1, b1, deg)


BQ = 512


def _mm2_body(h_ref, w_ref, deg_ref, o_ref, acc_ref):
    k = pl.program_id(1)

    @pl.when(k == 0)
    def _():
        acc_ref[...] = jnp.zeros_like(acc_ref)

    acc_ref[...] += jnp.dot(h_ref[...], w_ref[...],
                            preferred_element_type=jnp.float32)

    @pl.when(k == 3)
    def _():
        ns, _ = _norms(deg_ref)
        y = acc_ref[...] * ns[:, None]
        for c in range(BQ // FC):
            o_ref[c] = y[:, c * FC:(c + 1) * FC]


def _mm2(h1, W2q, deg):
    return pl.pallas_call(
        _mm2_body,
        grid=(NP // BN, 2048 // BQ),
        in_specs=[
            pl.BlockSpec((BN, BQ), lambda i, k: (i, k)),
            pl.BlockSpec((BQ, BQ), lambda i, k: (k, 0)),
            pl.BlockSpec((2, BN), lambda i, k: (0, i)),
        ],
        out_specs=pl.BlockSpec((BQ // FC, BN, FC), lambda i, k: (0, i, 0)),
        out_shape=jax.ShapeDtypeStruct((BQ // FC, NP, FC), jnp.float32),
        scratch_shapes=[pltpu.VMEM((BN, BQ), jnp.float32)],
    )(h1, W2q, deg)


def _tacc_body(t_ref, a_ref, w_ref, b_ref, deg_ref, o_ref):
    h = jnp.concatenate([a_ref[c] for c in range(BQ // FC)], axis=1)
    ns, nd = _norms(deg_ref)
    h2 = _leaky(h * nd[:, None] + b_ref[0, :][None, :])
    y = jnp.dot(h2, w_ref[...], preferred_element_type=jnp.float32)
    y = y * ns[:, None]
    o_ref[0] = t_ref[0] + y[:, :FC]
    o_ref[1] = t_ref[1] + y[:, FC:]


def _tacc(t, agg4, W3q, b2q, deg):
    return pl.pallas_call(
        _tacc_body,
        grid=(NP // BN,),
        in_specs=[
            pl.BlockSpec((2, BN, FC), lambda i: (0, i, 0)),
            pl.BlockSpec((BQ // FC, BN, FC), lambda i: (0, i, 0)),
            pl.BlockSpec((BQ, 2 * FC), lambda i: (0, 0)),
            pl.BlockSpec((1, BQ), lambda i: (0, 0)),
            pl.BlockSpec((2, BN), lambda i: (0, i)),
        ],
        out_specs=pl.BlockSpec((2, BN, FC), lambda i: (0, i, 0)),
        out_shape=jax.ShapeDtypeStruct((2, NP, FC), jnp.float32),
        input_output_aliases={0: 0},
    )(t, agg4, W3q, b2q, deg)


def _fin_body(a_ref, deg_ref, b_ref, o_ref):
    h = jnp.concatenate([a_ref[0], a_ref[1]], axis=1)
    _, nd = _norms(deg_ref)
    y = h * nd[:, None] + b_ref[...]
    nrm = jnp.sqrt(jnp.sum(y * y, axis=1, keepdims=True))
    o_ref[...] = y / jnp.clip(nrm, 1e-12, None)


def _fin(agg3, deg, b3):
    return pl.pallas_call(
        _fin_body,
        grid=(NP // BN,),
        in_specs=[
            pl.BlockSpec((2, BN, FC), lambda i: (0, i, 0)),
            pl.BlockSpec((2, BN), lambda i: (0, i)),
            pl.BlockSpec((1, 2 * FC), lambda i: (0, 0)),
        ],
        out_specs=pl.BlockSpec((BN, 2 * FC), lambda i: (i, 0)),
        out_shape=jax.ShapeDtypeStruct((NP, 2 * FC), jnp.float32),
    )(agg3, deg, b3)


def kernel(g, inputs, W1, b1, W2, b2, W3, b3):
    idx_all = _edge_arrays(g)
    x_pad = jnp.concatenate(
        [inputs, jnp.zeros((NP - N, 2 * FC), jnp.float32)])
    zeros2 = jnp.zeros((ZR, FC), jnp.float32)
    zeros1 = jnp.zeros((NROW, DW), jnp.float32)
    ones1 = jnp.ones((EB, DW), jnp.float32)

    deg = _degree(idx_all, ones1, zeros1)[:, :, 0]
    x1s = _scale(deg, x_pad)
    agg1 = _propagate(x1s, idx_all, zeros2)
    h1 = _m1(agg1, W1, b1.reshape(1, -1), deg)
    aggs = []
    for q in range(4):
        mmq = _mm2(h1, W2[:, q * BQ:(q + 1) * BQ], deg)
        aggs.append(_propagate(mmq, idx_all, zeros2))
    t = jnp.zeros((2, NP, FC), jnp.float32)
    for q in range(4):
        t = _tacc(t, aggs[q], W3[q * BQ:(q + 1) * BQ],
                  b2[q * BQ:(q + 1) * BQ].reshape(1, -1), deg)
    agg3 = _propagate(t, idx_all, zeros2)
    return _fin(agg3, deg, b3.reshape(1, -1))[:N]
